# baseline (speedup 1.0000x reference)
import functools

import jax
import jax.numpy as jnp
from jax import lax
from jax.experimental import pallas as pl
from jax.experimental.pallas import tpu as pltpu
from jax.experimental.pallas import tpu_sc as plsc

N_NODES_ = 10000
N_EDGES_ = 320000
K_PIECES = 5
N_E_P = N_EDGES_ // K_PIECES
EDGE_D = 16
NODE_D = 128
HID = 512
OUTD = 512

NC = 2
NS = 16
NW = NC * NS
EDGES_PER_W = N_E_P // NW
CHUNK = 80
NCHUNK = EDGES_PER_W // CHUNK
N_PAD = 10240
ROWS_PER_TILE = N_PAD // NS


def _sc_segment_sum(edge_c, recv_c, out_hbm, acc, idx_v, rows_a, rows_b,
                    zrow_v, sem_a, sem_b):
    cid = lax.axis_index("c")
    sid = lax.axis_index("s")
    wid = sid * NC + cid
    base = wid * NCHUNK

    z16 = jnp.zeros((16,), jnp.float32)

    def zbody(i, _):
        zrow_v[i] = z16
        return _

    lax.fori_loop(0, ROWS_PER_TILE, zbody, None)
    pltpu.sync_copy(zrow_v, acc.at[pl.ds(sid * ROWS_PER_TILE, ROWS_PER_TILE)])

    pltpu.sync_copy(recv_c.at[pl.ds(wid * EDGES_PER_W, EDGES_PER_W)], idx_v)
    plsc.subcore_barrier()

    def _chunk(j):
        return (0, pl.ds((base + j) * CHUNK, CHUNK))

    pltpu.make_async_copy(edge_c.at[_chunk(0)], rows_a, sem_a).start()

    def pair(p, _):
        ja = 2 * p
        pltpu.make_async_copy(edge_c.at[_chunk(ja)], rows_a, sem_a).wait()
        pltpu.make_async_copy(edge_c.at[_chunk(ja + 1)], rows_b, sem_b).start()
        pltpu.sync_copy(rows_a, acc.at[plsc.Indices(idx_v.at[pl.ds(ja * CHUNK, CHUNK)])], add=True)
        pltpu.make_async_copy(edge_c.at[_chunk(ja + 1)], rows_b, sem_b).wait()
        pltpu.make_async_copy(edge_c.at[_chunk(ja + 2)], rows_a, sem_a).start()
        pltpu.sync_copy(
            rows_b, acc.at[plsc.Indices(idx_v.at[pl.ds((ja + 1) * CHUNK, CHUNK)])],
            add=True)
        return _

    lax.fori_loop(0, (NCHUNK - 1) // 2, pair, None)
    pltpu.make_async_copy(edge_c.at[_chunk(NCHUNK - 1)], rows_a, sem_a).wait()
    pltpu.sync_copy(
        rows_a,
        acc.at[plsc.Indices(idx_v.at[pl.ds((NCHUNK - 1) * CHUNK, CHUNK)])],
        add=True)
    plsc.subcore_barrier()

    sl = pl.ds(sid * ROWS_PER_TILE, ROWS_PER_TILE)
    osl = pl.ds(cid * N_PAD + sid * ROWS_PER_TILE, ROWS_PER_TILE)
    pltpu.sync_copy(acc.at[sl], out_hbm.at[osl])


def _segment_sum_partials(edge_rows, receiver):
    edge_c = edge_rows
    recv_c = receiver
    mesh = plsc.VectorSubcoreMesh(core_axis_name="c", subcore_axis_name="s")
    fn = pl.kernel(
        _sc_segment_sum,
        out_type=jax.ShapeDtypeStruct((NC * N_PAD, EDGE_D), jnp.float32),
        mesh=mesh,
        compiler_params=pltpu.CompilerParams(use_tc_tiling_on_sc=False),
        scratch_types=[
            pltpu.VMEM_SHARED((N_PAD, EDGE_D), jnp.float32),
            pltpu.VMEM((EDGES_PER_W,), jnp.int32),
            pltpu.VMEM((CHUNK, EDGE_D), jnp.float32),
            pltpu.VMEM((CHUNK, EDGE_D), jnp.float32),
            pltpu.VMEM((ROWS_PER_TILE, EDGE_D), jnp.float32),
            pltpu.SemaphoreType.DMA,
            pltpu.SemaphoreType.DMA,
        ],
    )
    return fn(edge_c, recv_c).reshape(NC, N_PAD, EDGE_D)[:, :N_NODES_]


def _mlp1_body(node_ref, w1n_ref, b1_ref, h1_ref):
    h1_ref[...] = jnp.dot(node_ref[...], w1n_ref[...],
                          preferred_element_type=jnp.float32) + b1_ref[...]


def _mlp2_body(h1_ref, *rest):
    e_refs = rest[:2 * K_PIECES]
    w1e_ref, w2_ref, b2_ref, g_ref, bb_ref, out_ref = rest[2 * K_PIECES:]
    es = e_refs[0][...]
    for r in e_refs[1:]:
        es = es + r[...]
    h = h1_ref[...] + jnp.dot(es, w1e_ref[...],
                              preferred_element_type=jnp.float32)
    h = h * jax.nn.sigmoid(h)
    y = jnp.dot(h, w2_ref[...], preferred_element_type=jnp.float32)
    y = y + b2_ref[...]
    mu = jnp.mean(y, axis=-1, keepdims=True)
    yc = y - mu
    var = jnp.mean(yc * yc, axis=-1, keepdims=True)
    out_ref[...] = yc * lax.rsqrt(var + 1e-5) * g_ref[...] + bb_ref[...]


def _mlp1(node2d, w1n, b1):
    BM = 400
    grid = N_NODES_ // BM
    return pl.pallas_call(
        _mlp1_body,
        grid=(grid,),
        in_specs=[
            pl.BlockSpec((BM, NODE_D), lambda i: (i, 0)),
            pl.BlockSpec((NODE_D, HID), lambda i: (0, 0)),
            pl.BlockSpec((1, HID), lambda i: (0, 0)),
        ],
        out_specs=pl.BlockSpec((BM, HID), lambda i: (i, 0)),
        out_shape=jax.ShapeDtypeStruct((N_NODES_, HID), jnp.float32),
    )(node2d, w1n, b1)


def _mlp2(h1, e_list, w1e, w2, b2, g, bb):
    BM = 400
    grid = N_NODES_ // BM
    return pl.pallas_call(
        _mlp2_body,
        grid=(grid,),
        in_specs=[
            pl.BlockSpec((BM, HID), lambda i: (i, 0)),
        ] + [
            pl.BlockSpec((BM, EDGE_D), lambda i: (i, 0))
            for _ in range(2 * K_PIECES)
        ] + [
            pl.BlockSpec((EDGE_D, HID), lambda i: (0, 0)),
            pl.BlockSpec((HID, OUTD), lambda i: (0, 0)),
            pl.BlockSpec((1, OUTD), lambda i: (0, 0)),
            pl.BlockSpec((1, OUTD), lambda i: (0, 0)),
            pl.BlockSpec((1, OUTD), lambda i: (0, 0)),
        ],
        out_specs=pl.BlockSpec((BM, OUTD), lambda i: (i, 0)),
        out_shape=jax.ShapeDtypeStruct((N_NODES_, OUTD), jnp.float32),
    )(h1, *e_list, w1e, w2, b2, g, bb)


@jax.jit
def kernel(node, edge_index, edge_attr, W1, b1, W2, b2, ln_g, ln_b):
    b, n_nodes, node_d = node.shape
    receiver = edge_index[1]
    edge_rows = edge_attr

    e_list = []
    for k in range(K_PIECES):
        parts = _segment_sum_partials(
            edge_rows[:, k * N_E_P:(k + 1) * N_E_P],
            receiver[k * N_E_P:(k + 1) * N_E_P])
        e_list.append(parts[0])
        e_list.append(parts[1])

    w1n = W1[:, :NODE_D].T
    w1e = W1[:, NODE_D:].T
    h1 = _mlp1(node.reshape(n_nodes, node_d), w1n, b1.reshape(1, HID))
    out = _mlp2(h1, e_list, w1e, W2.T, b2.reshape(1, OUTD),
                ln_g.reshape(1, OUTD), ln_b.reshape(1, OUTD))
    return out.reshape(b, n_nodes, OUTD)


# speedup vs baseline: 1.3956x; 1.3956x over previous
import functools

import jax
import jax.numpy as jnp
from jax import lax
from jax.experimental import pallas as pl
from jax.experimental.pallas import tpu as pltpu
from jax.experimental.pallas import tpu_sc as plsc

N_NODES_ = 10000
N_EDGES_ = 320000
EDGE_D = 16
NODE_D = 128
HID = 512
OUTD = 512

NC = 2
NS = 16
NW = NC * NS
EDGES_PER_W = N_EDGES_ // NW
CHUNK = 80
NCHUNK = EDGES_PER_W // CHUNK
N_PAD = 10240
ROWS_PER_TILE = N_PAD // NS


def _sc_segment_sum(edge_c, recv_c, out_hbm, acc, idx_v, rows_a, rows_b,
                    zrow_v, sem_a, sem_b):
    cid = lax.axis_index("c")
    sid = lax.axis_index("s")
    wid = sid * NC + cid
    base = wid * NCHUNK

    z16 = jnp.zeros((16,), jnp.float32)

    def zbody(i, _):
        zrow_v[i] = z16
        return _

    lax.fori_loop(0, ROWS_PER_TILE, zbody, None)
    pltpu.sync_copy(zrow_v, acc.at[pl.ds(sid * ROWS_PER_TILE, ROWS_PER_TILE)])

    pltpu.sync_copy(recv_c.at[pl.ds(wid * EDGES_PER_W, EDGES_PER_W)], idx_v)
    plsc.subcore_barrier()

    def _chunk(j):
        return pl.ds((base + j) * CHUNK, CHUNK)

    pltpu.make_async_copy(edge_c.at[_chunk(0)], rows_a, sem_a).start()

    def pair(p, _):
        ja = 2 * p
        pltpu.make_async_copy(edge_c.at[_chunk(ja)], rows_a, sem_a).wait()
        pltpu.make_async_copy(edge_c.at[_chunk(ja + 1)], rows_b, sem_b).start()
        pltpu.sync_copy(rows_a, acc.at[plsc.Indices(idx_v.at[pl.ds(ja * CHUNK, CHUNK)])], add=True)
        pltpu.make_async_copy(edge_c.at[_chunk(ja + 1)], rows_b, sem_b).wait()
        pltpu.make_async_copy(edge_c.at[_chunk(ja + 2)], rows_a, sem_a).start()
        pltpu.sync_copy(
            rows_b, acc.at[plsc.Indices(idx_v.at[pl.ds((ja + 1) * CHUNK, CHUNK)])],
            add=True)
        return _

    lax.fori_loop(0, (NCHUNK - 1) // 2, pair, None)
    pltpu.make_async_copy(edge_c.at[_chunk(NCHUNK - 1)], rows_a, sem_a).wait()
    pltpu.sync_copy(
        rows_a,
        acc.at[plsc.Indices(idx_v.at[pl.ds((NCHUNK - 1) * CHUNK, CHUNK)])],
        add=True)
    plsc.subcore_barrier()

    sl = pl.ds(sid * ROWS_PER_TILE, ROWS_PER_TILE)
    osl = pl.ds(cid * N_PAD + sid * ROWS_PER_TILE, ROWS_PER_TILE)
    pltpu.sync_copy(acc.at[sl], out_hbm.at[osl])


def _segment_sum_partials(edge_rows, receiver):
    edge_c = edge_rows
    recv_c = receiver
    mesh = plsc.VectorSubcoreMesh(core_axis_name="c", subcore_axis_name="s")
    fn = pl.kernel(
        _sc_segment_sum,
        out_type=jax.ShapeDtypeStruct((NC * N_PAD, EDGE_D), jnp.float32),
        mesh=mesh,
        compiler_params=pltpu.CompilerParams(use_tc_tiling_on_sc=False),
        scratch_types=[
            pltpu.VMEM_SHARED((N_PAD, EDGE_D), jnp.float32),
            pltpu.VMEM((EDGES_PER_W,), jnp.int32),
            pltpu.VMEM((CHUNK, EDGE_D), jnp.float32),
            pltpu.VMEM((CHUNK, EDGE_D), jnp.float32),
            pltpu.VMEM((ROWS_PER_TILE, EDGE_D), jnp.float32),
            pltpu.SemaphoreType.DMA,
            pltpu.SemaphoreType.DMA,
        ],
    )
    return fn(edge_c, recv_c).reshape(NC, N_PAD, EDGE_D)[:, :N_NODES_]


def _mlp_body(node_ref, e0_ref, e1_ref, w1n_ref, w1e_ref, b1_ref, w2_ref,
              b2_ref, g_ref, bb_ref, out_ref):
    es = e0_ref[...] + e1_ref[...]
    h = jnp.dot(node_ref[...], w1n_ref[...],
                preferred_element_type=jnp.float32)
    h = h + jnp.dot(es, w1e_ref[...], preferred_element_type=jnp.float32)
    h = h + b1_ref[...]
    h = h * jax.nn.sigmoid(h)
    y = jnp.dot(h, w2_ref[...], preferred_element_type=jnp.float32)
    y = y + b2_ref[...]
    mu = jnp.mean(y, axis=-1, keepdims=True)
    yc = y - mu
    var = jnp.mean(yc * yc, axis=-1, keepdims=True)
    out_ref[...] = yc * lax.rsqrt(var + 1e-5) * g_ref[...] + bb_ref[...]


def _mlp(node2d, e0, e1, w1n, w1e, b1, w2, b2, g, bb):
    BM = 400
    grid = N_NODES_ // BM
    return pl.pallas_call(
        _mlp_body,
        grid=(grid,),
        in_specs=[
            pl.BlockSpec((BM, NODE_D), lambda i: (i, 0)),
            pl.BlockSpec((BM, EDGE_D), lambda i: (i, 0)),
            pl.BlockSpec((BM, EDGE_D), lambda i: (i, 0)),
            pl.BlockSpec((NODE_D, HID), lambda i: (0, 0)),
            pl.BlockSpec((EDGE_D, HID), lambda i: (0, 0)),
            pl.BlockSpec((1, HID), lambda i: (0, 0)),
            pl.BlockSpec((HID, OUTD), lambda i: (0, 0)),
            pl.BlockSpec((1, OUTD), lambda i: (0, 0)),
            pl.BlockSpec((1, OUTD), lambda i: (0, 0)),
            pl.BlockSpec((1, OUTD), lambda i: (0, 0)),
        ],
        out_specs=pl.BlockSpec((BM, OUTD), lambda i: (i, 0)),
        out_shape=jax.ShapeDtypeStruct((N_NODES_, OUTD), jnp.float32),
    )(node2d, e0, e1, w1n, w1e, b1, w2, b2, g, bb)


@jax.jit
def kernel(node, edge_index, edge_attr, W1, b1, W2, b2, ln_g, ln_b):
    b, n_nodes, node_d = node.shape
    receiver = edge_index[1]
    edge_rows = edge_attr.reshape(-1, EDGE_D)

    parts = _segment_sum_partials(edge_rows, receiver)
    e0 = parts[0]
    e1 = parts[1]

    w1n = W1[:, :NODE_D].T
    w1e = W1[:, NODE_D:].T
    out = _mlp(node.reshape(n_nodes, node_d), e0, e1, w1n, w1e,
               b1.reshape(1, HID), W2.T, b2.reshape(1, OUTD),
               ln_g.reshape(1, OUTD), ln_b.reshape(1, OUTD))
    return out.reshape(b, n_nodes, OUTD)


# speedup vs baseline: 1.5512x; 1.1116x over previous
import functools

import jax
import jax.numpy as jnp
from jax import lax
from jax.experimental import pallas as pl
from jax.experimental.pallas import tpu as pltpu
from jax.experimental.pallas import tpu_sc as plsc

N_NODES_ = 10000
N_EDGES_ = 320000
EDGE_D = 16
NODE_D = 128
HID = 512
OUTD = 512

NC = 2
NS = 16
NW = NC * NS
EDGES_PER_W = N_EDGES_ // NW
CHUNK = 80
NCHUNK = EDGES_PER_W // CHUNK
N_PAD = 10240
ROWS_PER_TILE = N_PAD // NS


def _sc_segment_sum(edge_c, recv_c, out_hbm, acc, idx_v, rows_a, rows_b,
                    zrow_v, sem_a, sem_b, sem_sa, sem_sb):
    cid = lax.axis_index("c")
    sid = lax.axis_index("s")
    wid = sid * NC + cid
    base = wid * NCHUNK

    z16 = jnp.zeros((16,), jnp.float32)

    def zbody(i, _):
        zrow_v[i] = z16
        return _

    lax.fori_loop(0, ROWS_PER_TILE, zbody, None)
    pltpu.sync_copy(zrow_v, acc.at[pl.ds(sid * ROWS_PER_TILE, ROWS_PER_TILE)])

    pltpu.sync_copy(recv_c.at[pl.ds(wid * EDGES_PER_W, EDGES_PER_W)], idx_v)
    plsc.subcore_barrier()

    def _chunk(j):
        return pl.ds((base + j) * CHUNK, CHUNK)

    npairs = (NCHUNK - 1) // 2

    def _scat_a(ja):
        return pltpu.make_async_copy(
            rows_a, acc.at[plsc.Indices(idx_v.at[pl.ds(ja * CHUNK, CHUNK)])],
            sem_sa)

    def _scat_b(jb):
        return pltpu.make_async_copy(
            rows_b, acc.at[plsc.Indices(idx_v.at[pl.ds(jb * CHUNK, CHUNK)])],
            sem_sb)

    pltpu.make_async_copy(edge_c.at[_chunk(0)], rows_a, sem_a).start()
    pltpu.make_async_copy(edge_c.at[_chunk(1)], rows_b, sem_b).start()

    def pair(p, _):
        ja = 2 * p
        pltpu.make_async_copy(edge_c.at[_chunk(ja)], rows_a, sem_a).wait()
        _scat_a(ja).start(add=True)
        pltpu.make_async_copy(edge_c.at[_chunk(ja + 1)], rows_b, sem_b).wait()
        _scat_b(ja + 1).start(add=True)
        _scat_a(ja).wait()
        pltpu.make_async_copy(edge_c.at[_chunk(ja + 2)], rows_a, sem_a).start()
        _scat_b(ja + 1).wait()

        @pl.when(p < npairs - 1)
        def _():
            pltpu.make_async_copy(edge_c.at[_chunk(ja + 3)], rows_b,
                                  sem_b).start()

        return _

    lax.fori_loop(0, npairs, pair, None)
    pltpu.make_async_copy(edge_c.at[_chunk(NCHUNK - 1)], rows_a, sem_a).wait()
    pltpu.sync_copy(
        rows_a,
        acc.at[plsc.Indices(idx_v.at[pl.ds((NCHUNK - 1) * CHUNK, CHUNK)])],
        add=True)
    plsc.subcore_barrier()

    sl = pl.ds(sid * ROWS_PER_TILE, ROWS_PER_TILE)
    osl = pl.ds(cid * N_PAD + sid * ROWS_PER_TILE, ROWS_PER_TILE)
    pltpu.sync_copy(acc.at[sl], out_hbm.at[osl])


def _segment_sum_partials(edge_rows, receiver):
    edge_c = edge_rows
    recv_c = receiver
    mesh = plsc.VectorSubcoreMesh(core_axis_name="c", subcore_axis_name="s")
    fn = pl.kernel(
        _sc_segment_sum,
        out_type=jax.ShapeDtypeStruct((NC * N_PAD, EDGE_D), jnp.float32),
        mesh=mesh,
        compiler_params=pltpu.CompilerParams(use_tc_tiling_on_sc=False),
        scratch_types=[
            pltpu.VMEM_SHARED((N_PAD, EDGE_D), jnp.float32),
            pltpu.VMEM((EDGES_PER_W,), jnp.int32),
            pltpu.VMEM((CHUNK, EDGE_D), jnp.float32),
            pltpu.VMEM((CHUNK, EDGE_D), jnp.float32),
            pltpu.VMEM((ROWS_PER_TILE, EDGE_D), jnp.float32),
            pltpu.SemaphoreType.DMA,
            pltpu.SemaphoreType.DMA,
            pltpu.SemaphoreType.DMA,
            pltpu.SemaphoreType.DMA,
        ],
    )
    return fn(edge_c, recv_c).reshape(NC, N_PAD, EDGE_D)[:, :N_NODES_]


def _mlp_body(node_ref, e0_ref, e1_ref, w1n_ref, w1e_ref, b1_ref, w2_ref,
              b2_ref, g_ref, bb_ref, out_ref):
    es = e0_ref[...] + e1_ref[...]
    h = jnp.dot(node_ref[...], w1n_ref[...],
                preferred_element_type=jnp.float32)
    h = h + jnp.dot(es, w1e_ref[...], preferred_element_type=jnp.float32)
    h = h + b1_ref[...]
    h = h * jax.nn.sigmoid(h)
    y = jnp.dot(h, w2_ref[...], preferred_element_type=jnp.float32)
    y = y + b2_ref[...]
    mu = jnp.mean(y, axis=-1, keepdims=True)
    yc = y - mu
    var = jnp.mean(yc * yc, axis=-1, keepdims=True)
    out_ref[...] = yc * lax.rsqrt(var + 1e-5) * g_ref[...] + bb_ref[...]


def _mlp(node2d, e0, e1, w1n, w1e, b1, w2, b2, g, bb):
    BM = 400
    grid = N_NODES_ // BM
    return pl.pallas_call(
        _mlp_body,
        grid=(grid,),
        in_specs=[
            pl.BlockSpec((BM, NODE_D), lambda i: (i, 0)),
            pl.BlockSpec((BM, EDGE_D), lambda i: (i, 0)),
            pl.BlockSpec((BM, EDGE_D), lambda i: (i, 0)),
            pl.BlockSpec((NODE_D, HID), lambda i: (0, 0)),
            pl.BlockSpec((EDGE_D, HID), lambda i: (0, 0)),
            pl.BlockSpec((1, HID), lambda i: (0, 0)),
            pl.BlockSpec((HID, OUTD), lambda i: (0, 0)),
            pl.BlockSpec((1, OUTD), lambda i: (0, 0)),
            pl.BlockSpec((1, OUTD), lambda i: (0, 0)),
            pl.BlockSpec((1, OUTD), lambda i: (0, 0)),
        ],
        out_specs=pl.BlockSpec((BM, OUTD), lambda i: (i, 0)),
        out_shape=jax.ShapeDtypeStruct((N_NODES_, OUTD), jnp.float32),
    )(node2d, e0, e1, w1n, w1e, b1, w2, b2, g, bb)


@jax.jit
def kernel(node, edge_index, edge_attr, W1, b1, W2, b2, ln_g, ln_b):
    b, n_nodes, node_d = node.shape
    receiver = edge_index[1]
    edge_rows = edge_attr.reshape(-1, EDGE_D)

    parts = _segment_sum_partials(edge_rows, receiver)
    e0 = parts[0]
    e1 = parts[1]

    w1n = W1[:, :NODE_D].T
    w1e = W1[:, NODE_D:].T
    out = _mlp(node.reshape(n_nodes, node_d), e0, e1, w1n, w1e,
               b1.reshape(1, HID), W2.T, b2.reshape(1, OUTD),
               ln_g.reshape(1, OUTD), ln_b.reshape(1, OUTD))
    return out.reshape(b, n_nodes, OUTD)
